# baseline (speedup 1.0000x reference)
import functools

import jax
import jax.numpy as jnp
from jax import lax
from jax.experimental import pallas as pl
from jax.experimental.pallas import tpu as pltpu
from jax.experimental.pallas import tpu_sc as plsc

B = 16384
N_SPARSE = 26
N_DENSE = 13
N_COLS = N_SPARSE + N_DENSE
VOCAB = 1000000

NC = 2
NS = 16
NW = NC * NS
RPW = B // NW
NCHUNK = RPW // 16
IDX_TOTAL = N_SPARSE * RPW
DMA_CHUNK = 128
DMA_PER_FIELD = RPW // DMA_CHUNK


def _body(inputs_hbm, table_hbm, w_hbm, out_hbm,
          in_v, idx_v, gath_v, out_v, w_v, sem):
    wid = lax.axis_index("s") * NC + lax.axis_index("c")
    base = pl.multiple_of(wid * RPW, RPW)

    fbase = pl.multiple_of(wid * (RPW * N_COLS), RPW * N_COLS)
    pltpu.sync_copy(inputs_hbm.at[pl.ds(fbase, RPW * N_COLS)], in_v)
    pltpu.sync_copy(w_hbm, w_v)

    def build_field(f, carry):
        off_f = pl.multiple_of(f * RPW, RPW)
        for c in range(NCHUNK):
            pos = (lax.iota(jnp.int32, 16) + c * 16) * N_COLS + f
            ids = plsc.load_gather(in_v, [pos])
            idx_v[pl.ds(off_f + c * 16, 16)] = ids + f * VOCAB
        for j in range(DMA_PER_FIELD):
            s = pl.ds(off_f + j * DMA_CHUNK, DMA_CHUNK)
            pltpu.async_copy(table_hbm.at[idx_v.at[s]], gath_v.at[s], sem)
        return carry

    lax.fori_loop(0, N_SPARSE, build_field, 0)

    pltpu.make_async_copy(table_hbm.at[pl.ds(0, IDX_TOTAL)], gath_v, sem).wait()

    def reduce_chunk(c, carry):
        c16 = pl.multiple_of(c * 16, 16)
        acc = gath_v[pl.ds(c16, 16)]
        for f in range(1, N_SPARSE):
            acc = acc + gath_v[pl.ds(f * RPW + c16, 16)]
        rowbase = (lax.iota(jnp.int32, 16) + c * 16) * N_COLS
        for d in range(N_DENSE):
            dv = plsc.load_gather(in_v, [rowbase + (N_SPARSE + d)])
            acc = acc + dv.astype(jnp.float32) * w_v[pl.ds(d * 16, 16)]
        out_v[pl.ds(c16, 16)] = acc
        return carry

    lax.fori_loop(0, NCHUNK, reduce_chunk, 0)

    pltpu.sync_copy(out_v, out_hbm.at[pl.ds(base, RPW)])


@functools.cache
def _sc_call():
    return pl.kernel(
        _body,
        mesh=plsc.VectorSubcoreMesh(
            core_axis_name="c", subcore_axis_name="s", num_cores=NC),
        out_type=jax.ShapeDtypeStruct((B,), jnp.float32),
        compiler_params=pltpu.CompilerParams(needs_layout_passes=False),
        scratch_types=[
            pltpu.VMEM((RPW * N_COLS,), jnp.int32),
            pltpu.VMEM((IDX_TOTAL,), jnp.int32),
            pltpu.VMEM((IDX_TOTAL,), jnp.float32),
            pltpu.VMEM((RPW,), jnp.float32),
            pltpu.VMEM((N_DENSE * 16,), jnp.float32),
            pltpu.SemaphoreType.DMA,
        ],
    )


def kernel(inputs, emb_tables, dense_weight):
    table_flat = emb_tables.reshape(-1)
    w_pad = jnp.broadcast_to(
        dense_weight.reshape(N_DENSE, 1), (N_DENSE, 16)).reshape(-1)
    out = _sc_call()(inputs.reshape(-1), table_flat, w_pad)
    return out.reshape(B, 1)


# speedup vs baseline: 43.7643x; 43.7643x over previous
import functools

import jax
import jax.numpy as jnp
from jax import lax
from jax.experimental import pallas as pl
from jax.experimental.pallas import tpu as pltpu
from jax.experimental.pallas import tpu_sc as plsc

B = 16384
N_SPARSE = 26
N_DENSE = 13
N_COLS = N_SPARSE + N_DENSE
VOCAB = 1000000

NC = 2
NS = 16
NW = NC * NS
RPW = B // NW
NCHUNK = RPW // 16
IDX_TOTAL = N_SPARSE * RPW
DMA_CHUNK = 128
DMA_PER_FIELD = RPW // DMA_CHUNK


def _body(inputs_hbm, table_hbm, w_hbm, out_hbm,
          in_v, idx_v, gath_v, out_v, w_v, sem):
    wid = lax.axis_index("s") * NC + lax.axis_index("c")
    base = pl.multiple_of(wid * RPW, RPW)

    fbase = pl.multiple_of(wid * (RPW * N_COLS), RPW * N_COLS)
    pltpu.sync_copy(inputs_hbm.at[pl.ds(fbase, RPW * N_COLS)], in_v)
    pltpu.sync_copy(w_hbm, w_v)

    def build_field(f, carry):
        off_f = pl.multiple_of(f * RPW, RPW)
        for c in range(NCHUNK):
            pos = (lax.iota(jnp.int32, 16) + c * 16) * N_COLS + f
            ids = plsc.load_gather(in_v, [pos])
            idx_v[pl.ds(off_f + c * 16, 16)] = ids
        src = table_hbm.at[f, 0]
        for j in range(DMA_PER_FIELD):
            s = pl.ds(off_f + j * DMA_CHUNK, DMA_CHUNK)
            pltpu.async_copy(src.at[idx_v.at[s]], gath_v.at[s], sem)
        return carry

    lax.fori_loop(0, N_SPARSE, build_field, 0)

    pltpu.make_async_copy(
        table_hbm.at[0, 0].at[pl.ds(0, IDX_TOTAL)], gath_v, sem).wait()

    def reduce_chunk(c, carry):
        c16 = pl.multiple_of(c * 16, 16)
        acc = gath_v[pl.ds(c16, 16)]
        for f in range(1, N_SPARSE):
            acc = acc + gath_v[pl.ds(f * RPW + c16, 16)]
        rowbase = (lax.iota(jnp.int32, 16) + c * 16) * N_COLS
        for d in range(N_DENSE):
            dv = plsc.load_gather(in_v, [rowbase + (N_SPARSE + d)])
            acc = acc + dv.astype(jnp.float32) * w_v[pl.ds(d * 16, 16)]
        out_v[pl.ds(c16, 16)] = acc
        return carry

    lax.fori_loop(0, NCHUNK, reduce_chunk, 0)

    pltpu.sync_copy(out_v, out_hbm.at[pl.ds(base, RPW)])


@functools.cache
def _sc_call():
    return pl.kernel(
        _body,
        mesh=plsc.VectorSubcoreMesh(
            core_axis_name="c", subcore_axis_name="s", num_cores=NC),
        out_type=jax.ShapeDtypeStruct((B,), jnp.float32),
        compiler_params=pltpu.CompilerParams(
            needs_layout_passes=False, use_tc_tiling_on_sc=True),
        scratch_types=[
            pltpu.VMEM((RPW * N_COLS,), jnp.int32),
            pltpu.VMEM((IDX_TOTAL,), jnp.int32),
            pltpu.VMEM((IDX_TOTAL,), jnp.float32),
            pltpu.VMEM((RPW,), jnp.float32),
            pltpu.VMEM((N_DENSE * 16,), jnp.float32),
            pltpu.SemaphoreType.DMA,
        ],
    )


def kernel(inputs, emb_tables, dense_weight):
    table3 = emb_tables.reshape(N_SPARSE, 1, VOCAB)
    w_pad = jnp.broadcast_to(
        dense_weight.reshape(N_DENSE, 1), (N_DENSE, 16)).reshape(-1)
    out = _sc_call()(inputs.reshape(-1), table3, w_pad)
    return out.reshape(B, 1)


# speedup vs baseline: 55.9666x; 1.2788x over previous
import functools

import jax
import jax.numpy as jnp
from jax import lax
from jax.experimental import pallas as pl
from jax.experimental.pallas import tpu as pltpu
from jax.experimental.pallas import tpu_sc as plsc

B = 16384
N_SPARSE = 26
N_DENSE = 13
N_COLS = N_SPARSE + N_DENSE
VOCAB = 1000000

NC = 2
NS = 16
NW = NC * NS
RPW = B // NW
NCHUNK = RPW // 16
IDX_TOTAL = N_SPARSE * RPW
DMA_CHUNK = 128
DMA_PER_FIELD = RPW // DMA_CHUNK


def _body(inputs_hbm, table_hbm, w_hbm, out_hbm,
          in_v, gath_v, out_v, w_v, sem, insem):
    wid = lax.axis_index("s") * NC + lax.axis_index("c")
    base = pl.multiple_of(wid * RPW, RPW)

    for col in range(N_COLS):
        pltpu.async_copy(
            inputs_hbm.at[pl.ds(col * B + base, RPW)],
            in_v.at[pl.ds(col * RPW, RPW)],
            insem)
    pltpu.sync_copy(w_hbm, w_v)
    pltpu.make_async_copy(
        inputs_hbm.at[pl.ds(0, N_COLS * RPW)], in_v, insem).wait()

    def build_field(f, carry):
        off_f = pl.multiple_of(f * RPW, RPW)
        src = table_hbm.at[f, 0]
        for j in range(DMA_PER_FIELD):
            s = pl.ds(off_f + j * DMA_CHUNK, DMA_CHUNK)
            pltpu.async_copy(src.at[in_v.at[s]], gath_v.at[s], sem)
        return carry

    lax.fori_loop(0, N_SPARSE, build_field, 0)

    pltpu.make_async_copy(
        table_hbm.at[0, 0].at[pl.ds(0, IDX_TOTAL)], gath_v, sem).wait()

    def reduce_chunk(c, carry):
        c16 = pl.multiple_of(c * 16, 16)
        acc = gath_v[pl.ds(c16, 16)]
        for f in range(1, N_SPARSE):
            acc = acc + gath_v[pl.ds(f * RPW + c16, 16)]
        for d in range(N_DENSE):
            dv = in_v[pl.ds((N_SPARSE + d) * RPW + c16, 16)]
            acc = acc + dv.astype(jnp.float32) * w_v[pl.ds(d * 16, 16)]
        out_v[pl.ds(c16, 16)] = acc
        return carry

    lax.fori_loop(0, NCHUNK, reduce_chunk, 0)

    pltpu.sync_copy(out_v, out_hbm.at[pl.ds(base, RPW)])


@functools.cache
def _sc_call():
    return pl.kernel(
        _body,
        mesh=plsc.VectorSubcoreMesh(
            core_axis_name="c", subcore_axis_name="s", num_cores=NC),
        out_type=jax.ShapeDtypeStruct((B,), jnp.float32),
        compiler_params=pltpu.CompilerParams(
            needs_layout_passes=False, use_tc_tiling_on_sc=True),
        scratch_types=[
            pltpu.VMEM((RPW * N_COLS,), jnp.int32),
            pltpu.VMEM((IDX_TOTAL,), jnp.float32),
            pltpu.VMEM((RPW,), jnp.float32),
            pltpu.VMEM((N_DENSE * 16,), jnp.float32),
            pltpu.SemaphoreType.DMA,
            pltpu.SemaphoreType.DMA,
        ],
    )


def kernel(inputs, emb_tables, dense_weight):
    in_flat = inputs.T.reshape(-1)
    table3 = emb_tables.reshape(N_SPARSE, 1, VOCAB)
    w_pad = jnp.broadcast_to(
        dense_weight.reshape(N_DENSE, 1), (N_DENSE, 16)).reshape(-1)
    out = _sc_call()(in_flat, table3, w_pad)
    return out.reshape(B, 1)


# speedup vs baseline: 55.9746x; 1.0001x over previous
import functools

import jax
import jax.numpy as jnp
from jax import lax
from jax.experimental import pallas as pl
from jax.experimental.pallas import tpu as pltpu
from jax.experimental.pallas import tpu_sc as plsc

B = 16384
N_SPARSE = 26
N_DENSE = 13
N_COLS = N_SPARSE + N_DENSE
VOCAB = 1000000

NC = 2
NS = 16
NW = NC * NS
RPW = B // NW
NCHUNK = RPW // 16
IDX_TOTAL = N_SPARSE * RPW
DMA_CHUNK = 512
DMA_PER_FIELD = RPW // DMA_CHUNK


def _body(inputs_hbm, table_hbm, w_hbm, out_hbm,
          in_v, gath_v, out_v, w_v, sem, insem):
    wid = lax.axis_index("s") * NC + lax.axis_index("c")
    base = pl.multiple_of(wid * RPW, RPW)

    for col in range(N_COLS):
        pltpu.async_copy(
            inputs_hbm.at[pl.ds(col * B + base, RPW)],
            in_v.at[pl.ds(col * RPW, RPW)],
            insem)
    pltpu.sync_copy(w_hbm, w_v)
    pltpu.make_async_copy(
        inputs_hbm.at[pl.ds(0, N_COLS * RPW)], in_v, insem).wait()

    def build_field(f, carry):
        off_f = pl.multiple_of(f * RPW, RPW)
        src = table_hbm.at[f, 0]
        for j in range(DMA_PER_FIELD):
            s = pl.ds(off_f + j * DMA_CHUNK, DMA_CHUNK)
            pltpu.async_copy(src.at[in_v.at[s]], gath_v.at[s], sem)
        return carry

    lax.fori_loop(0, N_SPARSE, build_field, 0)

    pltpu.make_async_copy(
        table_hbm.at[0, 0].at[pl.ds(0, IDX_TOTAL)], gath_v, sem).wait()

    def reduce_chunk(c, carry):
        c16 = pl.multiple_of(c * 16, 16)
        acc = gath_v[pl.ds(c16, 16)]
        for f in range(1, N_SPARSE):
            acc = acc + gath_v[pl.ds(f * RPW + c16, 16)]
        for d in range(N_DENSE):
            dv = in_v[pl.ds((N_SPARSE + d) * RPW + c16, 16)]
            acc = acc + dv.astype(jnp.float32) * w_v[pl.ds(d * 16, 16)]
        out_v[pl.ds(c16, 16)] = acc
        return carry

    lax.fori_loop(0, NCHUNK, reduce_chunk, 0)

    pltpu.sync_copy(out_v, out_hbm.at[pl.ds(base, RPW)])


@functools.cache
def _sc_call():
    return pl.kernel(
        _body,
        mesh=plsc.VectorSubcoreMesh(
            core_axis_name="c", subcore_axis_name="s", num_cores=NC),
        out_type=jax.ShapeDtypeStruct((B,), jnp.float32),
        compiler_params=pltpu.CompilerParams(
            needs_layout_passes=False, use_tc_tiling_on_sc=True),
        scratch_types=[
            pltpu.VMEM((RPW * N_COLS,), jnp.int32),
            pltpu.VMEM((IDX_TOTAL,), jnp.float32),
            pltpu.VMEM((RPW,), jnp.float32),
            pltpu.VMEM((N_DENSE * 16,), jnp.float32),
            pltpu.SemaphoreType.DMA,
            pltpu.SemaphoreType.DMA,
        ],
    )


def kernel(inputs, emb_tables, dense_weight):
    in_flat = inputs.T.reshape(-1)
    table3 = emb_tables.reshape(N_SPARSE, 1, VOCAB)
    w_pad = jnp.broadcast_to(
        dense_weight.reshape(N_DENSE, 1), (N_DENSE, 16)).reshape(-1)
    out = _sc_call()(in_flat, table3, w_pad)
    return out.reshape(B, 1)
